# baseline (speedup 1.0000x reference)
import functools
import math

import jax
import jax.numpy as jnp
from jax import lax
from jax.experimental import pallas as pl
from jax.experimental.pallas import tpu as pltpu
from jax.experimental.pallas import tpu_sc as plsc

EMB = 64
SCALE = math.sqrt(EMB)

B = 16384 * 50
NC, NS, L = 2, 16, 16
NW = NC * NS
PER_W = B // NW
C = 1024
IR = C // 128
CHUNKS = PER_W // C


def _body(tok_hbm, table_hbm, out_hbm, idx_v, rows_v, sem):
    wid = lax.axis_index("s") * NC + lax.axis_index("c")
    base = wid * PER_W

    def chunk(i, carry):
        row_off = base + i * C
        tok_off = pl.multiple_of(row_off // 128, IR)
        pltpu.sync_copy(tok_hbm.at[pl.ds(tok_off, IR)], idx_v)
        copies = []
        for j in range(IR):
            copies.append(pltpu.async_copy(
                table_hbm.at[idx_v.at[j]],
                rows_v.at[pl.ds(j * 128, 128)],
                sem))
        for cpy in copies:
            cpy.wait()

        def scale_row(r, c2):
            for c in range(EMB // L):
                rows_v[r, pl.ds(c * L, L)] = rows_v[r, pl.ds(c * L, L)] * SCALE
            return c2

        lax.fori_loop(0, C, scale_row, 0)
        pltpu.sync_copy(rows_v, out_hbm.at[pl.ds(row_off, C)])
        return carry

    lax.fori_loop(0, CHUNKS, chunk, 0)


@jax.jit
def _embed(tokens, table):
    tok2d = tokens.reshape(B // 128, 128).astype(jnp.int32)
    mesh = plsc.VectorSubcoreMesh(core_axis_name="c", subcore_axis_name="s")
    run = functools.partial(
        pl.kernel,
        out_type=jax.ShapeDtypeStruct((B, EMB), jnp.float32),
        mesh=mesh,
        scratch_types=[
            pltpu.VMEM((IR, 128), jnp.int32),
            pltpu.VMEM((C, EMB), jnp.float32),
            pltpu.SemaphoreType.DMA,
        ],
        compiler_params=pltpu.CompilerParams(use_tc_tiling_on_sc=False),
    )(_body)
    out = run(tok2d, table)
    return out.reshape(tokens.shape[0], tokens.shape[1], EMB)


def kernel(tokens, table):
    return _embed(tokens, table)


# speedup vs baseline: 1.0915x; 1.0915x over previous
import functools
import math

import jax
import jax.numpy as jnp
from jax import lax
from jax.experimental import pallas as pl
from jax.experimental.pallas import tpu as pltpu
from jax.experimental.pallas import tpu_sc as plsc

EMB = 64
SCALE = math.sqrt(EMB)

B = 16384 * 50
NC, NS, L = 2, 16, 16
NW = NC * NS
PER_W = B // NW
C = 512
IR = C // 128
NPAIR = PER_W // C // 2
UNROLL = 8


def _body(tok_hbm, table_hbm, out_hbm, idx0, idx1, r0, r1, g0, g1):
    wid = lax.axis_index("s") * NC + lax.axis_index("c")
    base = wid * PER_W

    def load_idx(idxb, c):
        off = pl.multiple_of((base + c * C) // 128, IR)
        pltpu.sync_copy(tok_hbm.at[pl.ds(off, IR)], idxb)

    def fire(idxb, rb, sem):
        for j in range(IR):
            pltpu.async_copy(table_hbm.at[idxb.at[j]],
                             rb.at[pl.ds(j * 128, 128)], sem)

    def drain(rb, sem):
        pltpu.make_async_copy(table_hbm.at[pl.ds(0, C)], rb, sem).wait()

    def scale(rb):
        def sbody(k, carry):
            r = k * UNROLL
            for rr in range(UNROLL):
                for c in range(EMB // L):
                    sl = pl.ds(c * L, L)
                    rb[r + rr, sl] = rb[r + rr, sl] * SCALE
            return carry

        lax.fori_loop(0, C // UNROLL, sbody, 0)

    def writeback(rb, c):
        pltpu.sync_copy(rb, out_hbm.at[pl.ds(base + c * C, C)])

    load_idx(idx0, 0)
    fire(idx0, r0, g0)

    def body(i, carry):
        c0 = 2 * i
        c1 = c0 + 1
        load_idx(idx1, c1)
        fire(idx1, r1, g1)
        drain(r0, g0)
        scale(r0)
        writeback(r0, c0)

        @pl.when(i < NPAIR - 1)
        def _():
            load_idx(idx0, c0 + 2)
            fire(idx0, r0, g0)

        drain(r1, g1)
        scale(r1)
        writeback(r1, c1)
        return carry

    lax.fori_loop(0, NPAIR, body, 0)


@jax.jit
def _embed(tokens, table):
    tok2d = tokens.reshape(B // 128, 128).astype(jnp.int32)
    mesh = plsc.VectorSubcoreMesh(core_axis_name="c", subcore_axis_name="s")
    run = functools.partial(
        pl.kernel,
        out_type=jax.ShapeDtypeStruct((B, EMB), jnp.float32),
        mesh=mesh,
        scratch_types=[
            pltpu.VMEM((IR, 128), jnp.int32),
            pltpu.VMEM((IR, 128), jnp.int32),
            pltpu.VMEM((C, EMB), jnp.float32),
            pltpu.VMEM((C, EMB), jnp.float32),
            pltpu.SemaphoreType.DMA,
            pltpu.SemaphoreType.DMA,
        ],
        compiler_params=pltpu.CompilerParams(use_tc_tiling_on_sc=False),
    )(_body)
    out = run(tok2d, table)
    return out.reshape(tokens.shape[0], tokens.shape[1], EMB)


def kernel(tokens, table):
    return _embed(tokens, table)


# speedup vs baseline: 1.1080x; 1.0151x over previous
import functools
import math

import jax
import jax.numpy as jnp
from jax import lax
from jax.experimental import pallas as pl
from jax.experimental.pallas import tpu as pltpu
from jax.experimental.pallas import tpu_sc as plsc

EMB = 64
SCALE = math.sqrt(EMB)

N0 = 16384
N1 = 50
B = N0 * N1
NC, NS, L = 2, 16, 16
NW = NC * NS
PER_W = B // NW
C = 512
NPAIR = PER_W // C // 2
UNROLL = 8


def _body(tok_hbm, table_hbm, out_hbm, idx_v, r0, r1, g0, g1):
    wid = lax.axis_index("s") * NC + lax.axis_index("c")
    base = wid * PER_W

    pltpu.sync_copy(tok_hbm.at[pl.ds(pl.multiple_of(base, 8), PER_W)], idx_v)

    def fire(rb, sem, c):
        off = c * C
        for j in range(C // 128):
            pltpu.async_copy(table_hbm.at[idx_v.at[pl.ds(off + j * 128, 128)]],
                             rb.at[pl.ds(j * 128, 128)], sem)

    def drain(rb, sem):
        pltpu.make_async_copy(table_hbm.at[pl.ds(0, C)], rb, sem).wait()

    def scale(rb):
        def sbody(k, carry):
            r = k * UNROLL
            for rr in range(UNROLL):
                for c in range(EMB // L):
                    sl = pl.ds(c * L, L)
                    rb[r + rr, sl] = rb[r + rr, sl] * SCALE
            return carry

        lax.fori_loop(0, C // UNROLL, sbody, 0)

    def writeback(rb, c):
        row = pl.multiple_of(base + c * C, 8)
        pltpu.sync_copy(rb, out_hbm.at[pl.ds(row, C)])

    fire(r0, g0, 0)

    def body(i, carry):
        c0 = 2 * i
        c1 = c0 + 1
        fire(r1, g1, c1)
        drain(r0, g0)
        scale(r0)
        writeback(r0, c0)

        @pl.when(i < NPAIR - 1)
        def _():
            fire(r0, g0, c0 + 2)

        drain(r1, g1)
        scale(r1)
        writeback(r1, c1)
        return carry

    lax.fori_loop(0, NPAIR, body, 0)


@jax.jit
def _embed(tokens, table):
    tok1d = tokens.reshape(B).astype(jnp.int32)
    mesh = plsc.VectorSubcoreMesh(core_axis_name="c", subcore_axis_name="s")
    run = functools.partial(
        pl.kernel,
        out_type=jax.ShapeDtypeStruct((B, EMB), jnp.float32),
        mesh=mesh,
        scratch_types=[
            pltpu.VMEM((PER_W,), jnp.int32),
            pltpu.VMEM((C, EMB), jnp.float32),
            pltpu.VMEM((C, EMB), jnp.float32),
            pltpu.SemaphoreType.DMA,
            pltpu.SemaphoreType.DMA,
        ],
        compiler_params=pltpu.CompilerParams(use_tc_tiling_on_sc=False),
    )(_body)
    return run(tok1d, table).reshape(N0, N1, EMB)


def kernel(tokens, table):
    return _embed(tokens, table)
